# Initial kernel scaffold; baseline (speedup 1.0000x reference)
import jax
import jax.numpy as jnp
from jax.experimental import pallas as pl


def kernel(x, batch, W1_0, b1_0, W2_0, b2_0, W3_0, b3_0, W1_1, b1_1, W2_1, b2_1, W3_1, b3_1, L1, bl1, L2, bl2, L3, bl3):
    raise NotImplementedError("write your pallas kernel here")


# speedup vs baseline: 14.8274x; 14.8274x over previous
import functools

import jax
import jax.numpy as jnp
from jax.experimental import pallas as pl
from jax.experimental.pallas import tpu as pltpu
from jax.experimental.pallas import tpu_sc as plsc

K_NN = 16
N_GRAPHS = 16


TILE_DST = 128
BLK_CAND = 512


def _knn_body(lo_ref, hi_ref, pcand_ref, bcand_ref, pdst_ref, bdst_ref,
              nbr_ref, d_ref, *, npad):
    t = pl.program_id(0)
    lo_b = lo_ref[t]
    hi_b = hi_ref[t]

    pd0 = pdst_ref[0:1, :]
    pd1 = pdst_ref[1:2, :]
    pd2 = pdst_ref[2:3, :]
    bd = bdst_ref[0:1, :]
    did = t * TILE_DST + jax.lax.broadcasted_iota(jnp.int32, (1, TILE_DST), 1)

    def dist_block(b, carry):
        base = b * BLK_CAND
        sl = pl.ds(base, BLK_CAND)
        dx0 = pcand_ref[sl, 0:1] - pd0
        dx1 = pcand_ref[sl, 1:2] - pd1
        dx2 = pcand_ref[sl, 2:3] - pd2
        d = dx0 * dx0 + dx1 * dx1 + dx2 * dx2
        bc = bcand_ref[sl, 0:1]
        cid = base + jax.lax.broadcasted_iota(jnp.int32, (BLK_CAND, 1), 0)
        mask = (bc != bd) | (cid == did)
        d_ref[sl, :] = jnp.where(mask, jnp.inf, d)
        return carry

    jax.lax.fori_loop(lo_b, hi_b, dist_block, 0)

    big = jnp.float32(2 ** 30)
    lastv = jnp.full((1, TILE_DST), -jnp.inf, jnp.float32)
    lasti = jnp.full((1, TILE_DST), -1.0, jnp.float32)
    picks = []
    for _ in range(K_NN):
        def scan_block(b, carry, lastv=lastv, lasti=lasti):
            runv, runi = carry
            base = b * BLK_CAND
            sl = pl.ds(base, BLK_CAND)
            db = d_ref[sl, :]
            cid = (base + jax.lax.broadcasted_iota(
                jnp.int32, (BLK_CAND, 1), 0)).astype(jnp.float32)
            elig = (db > lastv) | ((db == lastv) & (cid > lasti))
            de = jnp.where(elig, db, jnp.inf)
            m = jnp.min(de, axis=0, keepdims=True)
            ie = jnp.where(elig & (db == m), cid, big)
            im = jnp.min(ie, axis=0, keepdims=True)
            better = (m < runv) | ((m == runv) & (im < runi))
            return (jnp.where(better, m, runv),
                    jnp.where(better, im, runi))

        runv, runi = jax.lax.fori_loop(
            lo_b, hi_b, scan_block,
            (jnp.full((1, TILE_DST), jnp.inf, jnp.float32),
             jnp.full((1, TILE_DST), big, jnp.float32)))
        picks.append(runi)
        lastv, lasti = runv, runi

    nbr_ref[...] = jnp.concatenate(picks, axis=0).astype(jnp.int32)


def _knn_pallas(pcand, bcand, pdst, bdst, lo_blk, hi_blk, npad):
    nt = npad // TILE_DST
    body = functools.partial(_knn_body, npad=npad)
    return pl.pallas_call(
        body,
        grid=(nt,),
        in_specs=[
            pl.BlockSpec(memory_space=pltpu.SMEM),
            pl.BlockSpec(memory_space=pltpu.SMEM),
            pl.BlockSpec((npad, 3), lambda t: (0, 0)),
            pl.BlockSpec((npad, 1), lambda t: (0, 0)),
            pl.BlockSpec((3, TILE_DST), lambda t: (0, t)),
            pl.BlockSpec((1, TILE_DST), lambda t: (0, t)),
        ],
        out_specs=pl.BlockSpec((K_NN, TILE_DST), lambda t: (0, t)),
        out_shape=jax.ShapeDtypeStruct((K_NN, npad), jnp.int32),
        scratch_shapes=[pltpu.VMEM((npad, TILE_DST), jnp.float32)],
    )(lo_blk, hi_blk, pcand, bcand, pdst, bdst)


TILE_PROJ = 1024


def _proj_body(x_ref, wd_ref, wb_ref, b_ref, a_ref, c_ref):
    xv = x_ref[...]
    a_ref[...] = jnp.dot(xv, wd_ref[...],
                         preferred_element_type=jnp.float32,
                         precision=jax.lax.Precision.HIGHEST) + b_ref[...]
    c_ref[...] = jnp.dot(xv, wb_ref[...], preferred_element_type=jnp.float32,
                         precision=jax.lax.Precision.HIGHEST)


def _proj_pallas(x, wd, wb, b, npad):
    din, h = wd.shape
    return pl.pallas_call(
        _proj_body,
        grid=(npad // TILE_PROJ,),
        in_specs=[
            pl.BlockSpec((TILE_PROJ, din), lambda t: (t, 0)),
            pl.BlockSpec((din, h), lambda t: (0, 0)),
            pl.BlockSpec((din, h), lambda t: (0, 0)),
            pl.BlockSpec((1, h), lambda t: (0, 0)),
        ],
        out_specs=[
            pl.BlockSpec((TILE_PROJ, h), lambda t: (t, 0)),
            pl.BlockSpec((TILE_PROJ, h), lambda t: (t, 0)),
        ],
        out_shape=[
            jax.ShapeDtypeStruct((npad, h), jnp.float32),
            jax.ShapeDtypeStruct((npad, h), jnp.float32),
        ],
    )(x, wd, wb, b)


GATHER_WIN = 128


def _sc_gather(table, idx_flat):
    n_idx = idx_flat.shape[1]
    dim = table.shape[1]
    mesh = plsc.VectorSubcoreMesh(core_axis_name="core",
                                  subcore_axis_name="subcore")

    @functools.partial(
        pl.kernel,
        out_type=jax.ShapeDtypeStruct((n_idx, dim), jnp.float32),
        mesh=mesh)
    def gather_kernel(x_hbm, i_hbm, o_hbm):
        def body(i_vmem, o_vmem):
            pltpu.sync_copy(x_hbm.at[i_vmem.at[0]], o_vmem)

        pltpu.emit_pipeline(
            body,
            grid=(n_idx // GATHER_WIN,),
            in_specs=[pl.BlockSpec((1, GATHER_WIN), lambda i: (0, i))],
            out_specs=[pl.BlockSpec((GATHER_WIN, dim), lambda i: (i, 0))],
            core_axis_name=("core", "subcore"),
            dimension_semantics=(pltpu.PARALLEL,),
        )(i_hbm, o_hbm)

    return gather_kernel(table, idx_flat)


TILE_EDGE = 512


def _edge_body(a_ref, cg_ref, w2_ref, b2_ref, w3_ref, b3_ref, o_ref):
    ne = TILE_EDGE * K_NN
    a = a_ref[...]
    h = a.shape[1]
    arep = jnp.broadcast_to(a[:, None, :],
                            (TILE_EDGE, K_NN, h)).reshape(ne, h)
    hv = jnp.maximum(arep + cg_ref[...], 0.0)
    hv = jnp.maximum(
        jnp.dot(hv, w2_ref[...], preferred_element_type=jnp.float32,
                         precision=jax.lax.Precision.DEFAULT)
        + b2_ref[...], 0.0)
    m = jnp.dot(hv, w3_ref[...],
                preferred_element_type=jnp.float32,
                         precision=jax.lax.Precision.DEFAULT) + b3_ref[...]
    dout = m.shape[1]
    mm = jnp.max(m.reshape(TILE_EDGE, K_NN, dout), axis=1)
    o_ref[...] = jnp.maximum(mm, 0.0)


def _edge_pallas(a, cg, w2, b2, w3, b3, npad):
    h = w2.shape[0]
    dout = w3.shape[1]
    ne = TILE_EDGE * K_NN
    return pl.pallas_call(
        _edge_body,
        grid=(npad // TILE_EDGE,),
        in_specs=[
            pl.BlockSpec((TILE_EDGE, h), lambda t: (t, 0)),
            pl.BlockSpec((ne, h), lambda t: (t, 0)),
            pl.BlockSpec((h, h), lambda t: (0, 0)),
            pl.BlockSpec((1, h), lambda t: (0, 0)),
            pl.BlockSpec((h, dout), lambda t: (0, 0)),
            pl.BlockSpec((1, dout), lambda t: (0, 0)),
        ],
        out_specs=pl.BlockSpec((TILE_EDGE, dout), lambda t: (t, 0)),
        out_shape=jax.ShapeDtypeStruct((npad, dout), jnp.float32),
    )(a, cg, w2, b2, w3, b3)


TILE_POOL = 1024


def _pool_body(h_ref, bcol_ref, l1_ref, bl1_ref, l2_ref, bl2_ref, l3_ref,
               bl3_ref, o_ref, sum_ref, cnt_ref):
    t = pl.program_id(0)

    @pl.when(t == 0)
    def _():
        sum_ref[...] = jnp.zeros_like(sum_ref)
        cnt_ref[...] = jnp.zeros_like(cnt_ref)

    bc = bcol_ref[...]
    g = jax.lax.broadcasted_iota(jnp.int32, (N_GRAPHS, 1), 0)
    oh = (bc == g).astype(jnp.float32)
    sum_ref[...] += jnp.dot(oh, h_ref[...],
                            preferred_element_type=jnp.float32,
                         precision=jax.lax.Precision.DEFAULT)
    cnt_ref[...] += jnp.sum(oh, axis=1, keepdims=True)

    @pl.when(t == pl.num_programs(0) - 1)
    def _():
        pooled = sum_ref[...] / jnp.maximum(cnt_ref[...], 1.0)
        o1 = jnp.maximum(
            jnp.dot(pooled, l1_ref[...], preferred_element_type=jnp.float32,
                         precision=jax.lax.Precision.DEFAULT)
            + bl1_ref[...], 0.0)
        o2 = jnp.maximum(
            jnp.dot(o1, l2_ref[...], preferred_element_type=jnp.float32,
                         precision=jax.lax.Precision.DEFAULT)
            + bl2_ref[...], 0.0)
        o_ref[...] = jnp.dot(o2, l3_ref[...],
                             preferred_element_type=jnp.float32,
                         precision=jax.lax.Precision.DEFAULT) + bl3_ref[...]


def _pool_pallas(hfeat, bcol, l1, bl1, l2, bl2, l3, bl3, npad):
    lat = hfeat.shape[1]
    return pl.pallas_call(
        _pool_body,
        grid=(npad // TILE_POOL,),
        in_specs=[
            pl.BlockSpec((TILE_POOL, lat), lambda t: (t, 0)),
            pl.BlockSpec((1, TILE_POOL), lambda t: (0, t)),
            pl.BlockSpec((lat, lat), lambda t: (0, 0)),
            pl.BlockSpec((1, lat), lambda t: (0, 0)),
            pl.BlockSpec((lat, lat), lambda t: (0, 0)),
            pl.BlockSpec((1, lat), lambda t: (0, 0)),
            pl.BlockSpec((lat, 1), lambda t: (0, 0)),
            pl.BlockSpec((1, 1), lambda t: (0, 0)),
        ],
        out_specs=pl.BlockSpec((N_GRAPHS, 1), lambda t: (0, 0)),
        out_shape=jax.ShapeDtypeStruct((N_GRAPHS, 1), jnp.float32),
        scratch_shapes=[pltpu.VMEM((N_GRAPHS, lat), jnp.float32),
                        pltpu.VMEM((N_GRAPHS, 1), jnp.float32)],
    )(hfeat, bcol, l1, bl1, l2, bl2, l3, bl3)


def kernel(x, batch, W1_0, b1_0, W2_0, b2_0, W3_0, b3_0,
           W1_1, b1_1, W2_1, b2_1, W3_1, b3_1, L1, bl1, L2, bl2, L3, bl3):
    n, dfeat = x.shape
    npad = ((n + TILE_PROJ - 1) // TILE_PROJ) * TILE_PROJ
    batch = batch.astype(jnp.int32)

    pos = x[:, :3]
    pcand = jnp.concatenate(
        [pos, jnp.zeros((npad - n, 3), jnp.float32)], axis=0)
    pdst = pcand.T
    bcand = jnp.concatenate(
        [batch, jnp.full((npad - n,), -1, jnp.int32)])[:, None]
    brow = jnp.concatenate(
        [batch, jnp.full((npad - n,), N_GRAPHS - 1, jnp.int32)])
    bdst = brow[None, :]
    bcol = bcand[:, 0][None, :]

    seg = jnp.searchsorted(
        batch, jnp.arange(N_GRAPHS + 1, dtype=jnp.int32),
        side="left").astype(jnp.int32)
    sizes = seg[1:] - seg[:-1]
    nt = npad // TILE_DST
    tidx = jnp.arange(nt, dtype=jnp.int32)
    bfirst = brow[tidx * TILE_DST]
    blast = brow[tidx * TILE_DST + TILE_DST - 1]
    gidx = jnp.arange(N_GRAPHS, dtype=jnp.int32)
    inrange = (gidx[None, :] >= bfirst[:, None]) & \
              (gidx[None, :] <= blast[:, None])
    minsz = jnp.min(jnp.where(inrange, sizes[None, :], n + 1), axis=1)
    degen = minsz < K_NN + 1
    lo = jnp.where(degen, 0, seg[bfirst])
    hi = jnp.where(degen, n, seg[blast + 1])
    lo_blk = lo // BLK_CAND
    hi_blk = (hi + BLK_CAND - 1) // BLK_CAND

    nbr_t = _knn_pallas(pcand, bcand, pdst, bdst, lo_blk, hi_blk, npad)
    idx_flat = nbr_t.T.reshape(1, npad * K_NN)

    feat = jnp.concatenate(
        [x, jnp.zeros((npad - n, dfeat), jnp.float32)], axis=0)
    for (w1, bb1, w2, bb2, w3, bb3) in (
            (W1_0, b1_0, W2_0, b2_0, W3_0, b3_0),
            (W1_1, b1_1, W2_1, b2_1, W3_1, b3_1)):
        din = feat.shape[1]
        wa, wb = w1[:din], w1[din:]
        av, cv = _proj_pallas(feat, wa - wb, wb, bb1[None, :], npad)
        cg = _sc_gather(cv, idx_flat)
        feat = _edge_pallas(av, cg, w2, bb2[None, :], w3, bb3[None, :], npad)

    return _pool_pallas(feat, bcol, L1, bl1[None, :], L2, bl2[None, :],
                        L3, bl3.reshape(1, 1), npad)


# speedup vs baseline: 18.3796x; 1.2396x over previous
import functools

import jax
import jax.numpy as jnp
from jax.experimental import pallas as pl
from jax.experimental.pallas import tpu as pltpu
from jax.experimental.pallas import tpu_sc as plsc

K_NN = 16
N_GRAPHS = 16

INF_KEY = 0x7F800000
DEAD_KEY = 0x7FFFFFFF


TILE_DST = 128
BLK_CAND = 512


def _knn_body(lo_ref, hi_ref, pcand_ref, bcand_ref, pdst_ref, bdst_ref,
              x_ref, wd_ref, wb_ref, b1_ref, nbr_ref, a_ref, c_ref, k_ref):
    t = pl.program_id(0)
    lo_b = lo_ref[t]
    hi_b = hi_ref[t]

    xv = x_ref[...]
    a_ref[...] = jnp.dot(xv, wd_ref[...], preferred_element_type=jnp.float32,
                         precision=jax.lax.Precision.HIGHEST) + b1_ref[...]
    c_ref[...] = jnp.dot(xv, wb_ref[...], preferred_element_type=jnp.float32,
                         precision=jax.lax.Precision.HIGHEST)

    pd0 = pdst_ref[0:1, :]
    pd1 = pdst_ref[1:2, :]
    pd2 = pdst_ref[2:3, :]
    bd = bdst_ref[0:1, :]
    did = t * TILE_DST + jax.lax.broadcasted_iota(jnp.int32, (1, TILE_DST), 1)

    def dist_block(b, carry):
        base = b * BLK_CAND
        sl = pl.ds(base, BLK_CAND)
        dx0 = pcand_ref[sl, 0:1] - pd0
        dx1 = pcand_ref[sl, 1:2] - pd1
        dx2 = pcand_ref[sl, 2:3] - pd2
        d = dx0 * dx0 + dx1 * dx1 + dx2 * dx2
        bc = bcand_ref[sl, 0:1]
        cid = base + jax.lax.broadcasted_iota(jnp.int32, (BLK_CAND, 1), 0)
        mask = (bc != bd) | (cid == did)
        key = jax.lax.bitcast_convert_type(d, jnp.int32)
        k_ref[sl, :] = jnp.where(mask, INF_KEY, key)
        return carry

    jax.lax.fori_loop(lo_b, hi_b, dist_block, 0)

    lasti = jnp.full((1, TILE_DST), -1, jnp.int32)
    picks = []
    for it in range(K_NN):
        last = it == K_NN - 1

        def scan_block(b, carry, lasti=lasti, last=last):
            runm, runi = carry
            base = b * BLK_CAND
            sl = pl.ds(base, BLK_CAND)
            cid = base + jax.lax.broadcasted_iota(
                jnp.int32, (BLK_CAND, 1), 0)
            kb = k_ref[sl, :]
            kb = jnp.where(cid == lasti, DEAD_KEY, kb)
            if not last:
                k_ref[sl, :] = kb
            m = jnp.min(kb, axis=0, keepdims=True)
            ie = jnp.where(kb == m, cid, jnp.int32(2 ** 30))
            im = jnp.min(ie, axis=0, keepdims=True)
            better = (m < runm) | ((m == runm) & (im < runi))
            return (jnp.where(better, m, runm),
                    jnp.where(better, im, runi))

        runm, runi = jax.lax.fori_loop(
            lo_b, hi_b, scan_block,
            (jnp.full((1, TILE_DST), DEAD_KEY, jnp.int32),
             jnp.full((1, TILE_DST), 2 ** 30, jnp.int32)))
        picks.append(runi)
        lasti = runi

    nbr_ref[...] = jnp.concatenate(picks, axis=0)


def _knn_pallas(pcand, bcand, pdst, bdst, xp, wd, wb, b1, lo_blk, hi_blk,
                npad):
    nt = npad // TILE_DST
    dfeat = xp.shape[1]
    h = wd.shape[1]
    return pl.pallas_call(
        _knn_body,
        grid=(nt,),
        in_specs=[
            pl.BlockSpec(memory_space=pltpu.SMEM),
            pl.BlockSpec(memory_space=pltpu.SMEM),
            pl.BlockSpec((npad, 3), lambda t: (0, 0)),
            pl.BlockSpec((npad, 1), lambda t: (0, 0)),
            pl.BlockSpec((3, TILE_DST), lambda t: (0, t)),
            pl.BlockSpec((1, TILE_DST), lambda t: (0, t)),
            pl.BlockSpec((TILE_DST, dfeat), lambda t: (t, 0)),
            pl.BlockSpec((dfeat, h), lambda t: (0, 0)),
            pl.BlockSpec((dfeat, h), lambda t: (0, 0)),
            pl.BlockSpec((1, h), lambda t: (0, 0)),
        ],
        out_specs=[
            pl.BlockSpec((K_NN, TILE_DST), lambda t: (0, t)),
            pl.BlockSpec((TILE_DST, h), lambda t: (t, 0)),
            pl.BlockSpec((TILE_DST, h), lambda t: (t, 0)),
        ],
        out_shape=[
            jax.ShapeDtypeStruct((K_NN, npad), jnp.int32),
            jax.ShapeDtypeStruct((npad, h), jnp.float32),
            jax.ShapeDtypeStruct((npad, h), jnp.float32),
        ],
        scratch_shapes=[pltpu.VMEM((npad, TILE_DST), jnp.int32)],
    )(lo_blk, hi_blk, pcand, bcand, pdst, bdst, xp, wd, wb, b1)


GATHER_WIN = 128


def _sc_gather(table, idx_flat):
    n_idx = idx_flat.shape[1]
    dim = table.shape[1]
    mesh = plsc.VectorSubcoreMesh(core_axis_name="core",
                                  subcore_axis_name="subcore")

    @functools.partial(
        pl.kernel,
        out_type=jax.ShapeDtypeStruct((n_idx, dim), jnp.float32),
        mesh=mesh)
    def gather_kernel(x_hbm, i_hbm, o_hbm):
        def body(i_vmem, o_vmem):
            pltpu.sync_copy(x_hbm.at[i_vmem.at[0]], o_vmem)

        pltpu.emit_pipeline(
            body,
            grid=(n_idx // GATHER_WIN,),
            in_specs=[pl.BlockSpec((1, GATHER_WIN), lambda i: (0, i))],
            out_specs=[pl.BlockSpec((GATHER_WIN, dim), lambda i: (i, 0))],
            core_axis_name=("core", "subcore"),
            dimension_semantics=(pltpu.PARALLEL,),
        )(i_hbm, o_hbm)

    return gather_kernel(table, idx_flat)


TILE_EDGE = 512


def _edge_mlp(a, cg, w2_ref, b2_ref, w3_ref, b3_ref):
    h = a.shape[1]
    ne = TILE_EDGE * K_NN
    arep = jnp.broadcast_to(a[:, None, :], (TILE_EDGE, K_NN, h)).reshape(ne, h)
    hv = jnp.maximum(arep + cg, 0.0)
    hv = jnp.maximum(
        jnp.dot(hv, w2_ref[...], preferred_element_type=jnp.float32,
                precision=jax.lax.Precision.DEFAULT) + b2_ref[...], 0.0)
    m = jnp.dot(hv, w3_ref[...], preferred_element_type=jnp.float32,
                precision=jax.lax.Precision.DEFAULT) + b3_ref[...]
    dout = m.shape[1]
    mm = jnp.max(m.reshape(TILE_EDGE, K_NN, dout), axis=1)
    return jnp.maximum(mm, 0.0)


def _edge0_body(a_ref, cg_ref, w2_ref, b2_ref, w3_ref, b3_ref,
                wd_ref, wb_ref, b1_ref, a1_ref, c1_ref):
    hv = _edge_mlp(a_ref[...], cg_ref[...], w2_ref, b2_ref, w3_ref, b3_ref)
    a1_ref[...] = jnp.dot(hv, wd_ref[...], preferred_element_type=jnp.float32,
                          precision=jax.lax.Precision.HIGHEST) + b1_ref[...]
    c1_ref[...] = jnp.dot(hv, wb_ref[...], preferred_element_type=jnp.float32,
                          precision=jax.lax.Precision.HIGHEST)


def _edge0_pallas(a, cg, w2, b2, w3, b3, wd, wb, b1, npad):
    h = w2.shape[0]
    dout = w3.shape[1]
    h1 = wd.shape[1]
    ne = TILE_EDGE * K_NN
    return pl.pallas_call(
        _edge0_body,
        grid=(npad // TILE_EDGE,),
        in_specs=[
            pl.BlockSpec((TILE_EDGE, h), lambda t: (t, 0)),
            pl.BlockSpec((ne, h), lambda t: (t, 0)),
            pl.BlockSpec((h, h), lambda t: (0, 0)),
            pl.BlockSpec((1, h), lambda t: (0, 0)),
            pl.BlockSpec((h, dout), lambda t: (0, 0)),
            pl.BlockSpec((1, dout), lambda t: (0, 0)),
            pl.BlockSpec((dout, h1), lambda t: (0, 0)),
            pl.BlockSpec((dout, h1), lambda t: (0, 0)),
            pl.BlockSpec((1, h1), lambda t: (0, 0)),
        ],
        out_specs=[
            pl.BlockSpec((TILE_EDGE, h1), lambda t: (t, 0)),
            pl.BlockSpec((TILE_EDGE, h1), lambda t: (t, 0)),
        ],
        out_shape=[
            jax.ShapeDtypeStruct((npad, h1), jnp.float32),
            jax.ShapeDtypeStruct((npad, h1), jnp.float32),
        ],
    )(a, cg, w2, b2, w3, b3, wd, wb, b1)


def _edge1_body(a_ref, cg_ref, w2_ref, b2_ref, w3_ref, b3_ref, bcol_ref,
                l1_ref, bl1_ref, l2_ref, bl2_ref, l3_ref, bl3_ref,
                o_ref, sum_ref, cnt_ref):
    t = pl.program_id(0)

    @pl.when(t == 0)
    def _():
        sum_ref[...] = jnp.zeros_like(sum_ref)
        cnt_ref[...] = jnp.zeros_like(cnt_ref)

    hv = _edge_mlp(a_ref[...], cg_ref[...], w2_ref, b2_ref, w3_ref, b3_ref)

    bc = bcol_ref[...]
    g = jax.lax.broadcasted_iota(jnp.int32, (N_GRAPHS, 1), 0)
    oh = (bc == g).astype(jnp.float32)
    sum_ref[...] += jnp.dot(oh, hv, preferred_element_type=jnp.float32,
                            precision=jax.lax.Precision.DEFAULT)
    cnt_ref[...] += jnp.sum(oh, axis=1, keepdims=True)

    @pl.when(t == pl.num_programs(0) - 1)
    def _():
        pooled = sum_ref[...] / jnp.maximum(cnt_ref[...], 1.0)
        o1 = jnp.maximum(
            jnp.dot(pooled, l1_ref[...], preferred_element_type=jnp.float32,
                    precision=jax.lax.Precision.DEFAULT) + bl1_ref[...], 0.0)
        o2 = jnp.maximum(
            jnp.dot(o1, l2_ref[...], preferred_element_type=jnp.float32,
                    precision=jax.lax.Precision.DEFAULT) + bl2_ref[...], 0.0)
        o_ref[...] = jnp.dot(o2, l3_ref[...],
                             preferred_element_type=jnp.float32,
                             precision=jax.lax.Precision.DEFAULT) + bl3_ref[...]


def _edge1_pallas(a, cg, w2, b2, w3, b3, bcol, l1, bl1, l2, bl2, l3, bl3,
                  npad):
    h = w2.shape[0]
    dout = w3.shape[1]
    ne = TILE_EDGE * K_NN
    return pl.pallas_call(
        _edge1_body,
        grid=(npad // TILE_EDGE,),
        in_specs=[
            pl.BlockSpec((TILE_EDGE, h), lambda t: (t, 0)),
            pl.BlockSpec((ne, h), lambda t: (t, 0)),
            pl.BlockSpec((h, h), lambda t: (0, 0)),
            pl.BlockSpec((1, h), lambda t: (0, 0)),
            pl.BlockSpec((h, dout), lambda t: (0, 0)),
            pl.BlockSpec((1, dout), lambda t: (0, 0)),
            pl.BlockSpec((1, TILE_EDGE), lambda t: (0, t)),
            pl.BlockSpec((dout, dout), lambda t: (0, 0)),
            pl.BlockSpec((1, dout), lambda t: (0, 0)),
            pl.BlockSpec((dout, dout), lambda t: (0, 0)),
            pl.BlockSpec((1, dout), lambda t: (0, 0)),
            pl.BlockSpec((dout, 1), lambda t: (0, 0)),
            pl.BlockSpec((1, 1), lambda t: (0, 0)),
        ],
        out_specs=pl.BlockSpec((N_GRAPHS, 1), lambda t: (0, 0)),
        out_shape=jax.ShapeDtypeStruct((N_GRAPHS, 1), jnp.float32),
        scratch_shapes=[pltpu.VMEM((N_GRAPHS, dout), jnp.float32),
                        pltpu.VMEM((N_GRAPHS, 1), jnp.float32)],
    )(a, cg, w2, b2, w3, b3, bcol, l1, bl1, l2, bl2, l3, bl3)


def kernel(x, batch, W1_0, b1_0, W2_0, b2_0, W3_0, b3_0,
           W1_1, b1_1, W2_1, b2_1, W3_1, b3_1, L1, bl1, L2, bl2, L3, bl3):
    n, dfeat = x.shape
    npad = ((n + TILE_EDGE - 1) // TILE_EDGE) * TILE_EDGE
    lat = W3_0.shape[1]
    batch = batch.astype(jnp.int32)

    pos = x[:, :3]
    pcand = jnp.concatenate(
        [pos, jnp.zeros((npad - n, 3), jnp.float32)], axis=0)
    pdst = pcand.T
    bcand = jnp.concatenate(
        [batch, jnp.full((npad - n,), -1, jnp.int32)])[:, None]
    brow = jnp.concatenate(
        [batch, jnp.full((npad - n,), N_GRAPHS - 1, jnp.int32)])
    bdst = brow[None, :]
    bcol = bcand[:, 0][None, :]

    seg = jnp.searchsorted(
        batch, jnp.arange(N_GRAPHS + 1, dtype=jnp.int32),
        side="left").astype(jnp.int32)
    sizes = seg[1:] - seg[:-1]
    nt = npad // TILE_DST
    tidx = jnp.arange(nt, dtype=jnp.int32)
    bfirst = brow[tidx * TILE_DST]
    blast = brow[tidx * TILE_DST + TILE_DST - 1]
    gidx = jnp.arange(N_GRAPHS, dtype=jnp.int32)
    inrange = (gidx[None, :] >= bfirst[:, None]) & \
              (gidx[None, :] <= blast[:, None])
    minsz = jnp.min(jnp.where(inrange, sizes[None, :], n + 1), axis=1)
    degen = minsz < K_NN + 1
    lo = jnp.where(degen, 0, seg[bfirst])
    hi = jnp.where(degen, n, seg[blast + 1])
    lo_blk = lo // BLK_CAND
    hi_blk = (hi + BLK_CAND - 1) // BLK_CAND

    xp = jnp.concatenate(
        [x, jnp.zeros((npad - n, dfeat), jnp.float32)], axis=0)

    wa0, wb0 = W1_0[:dfeat], W1_0[dfeat:]
    nbr_t, a0, c0 = _knn_pallas(pcand, bcand, pdst, bdst, xp,
                                wa0 - wb0, wb0, b1_0[None, :],
                                lo_blk, hi_blk, npad)
    idx_flat = nbr_t.T.reshape(1, npad * K_NN)

    cg0 = _sc_gather(c0, idx_flat)
    wa1, wb1 = W1_1[:lat], W1_1[lat:]
    a1, c1 = _edge0_pallas(a0, cg0, W2_0, b2_0[None, :], W3_0, b3_0[None, :],
                           wa1 - wb1, wb1, b1_1[None, :], npad)

    cg1 = _sc_gather(c1, idx_flat)
    return _edge1_pallas(a1, cg1, W2_1, b2_1[None, :], W3_1, b3_1[None, :],
                         bcol, L1, bl1[None, :], L2, bl2[None, :],
                         L3, bl3.reshape(1, 1), npad)
